# Initial kernel scaffold; baseline (speedup 1.0000x reference)
import jax
import jax.numpy as jnp
from jax.experimental import pallas as pl


def kernel(ids, emb_table):
    raise NotImplementedError("write your pallas kernel here")


# speedup vs baseline: 1.1430x; 1.1430x over previous
import functools

import jax
import jax.numpy as jnp
from jax import lax
from jax.experimental import pallas as pl
from jax.experimental.pallas import tpu as pltpu
from jax.experimental.pallas import tpu_sc as plsc

NC = 2
NS = 16
NW = NC * NS


def _emb_kernel(B, D, CHUNK):
  n_per_w = B // NW
  n_chunks = n_per_w // CHUNK
  mesh = plsc.VectorSubcoreMesh(
      core_axis_name="c", subcore_axis_name="s", num_cores=NC,
      num_subcores=NS)

  @functools.partial(
      pl.kernel,
      mesh=mesh,
      compiler_params=pltpu.CompilerParams(use_tc_tiling_on_sc=False),
      out_type=jax.ShapeDtypeStruct((B, D), jnp.float32),
      scratch_types=[
          pltpu.VMEM((CHUNK,), jnp.int32),
          pltpu.VMEM((CHUNK, D), jnp.float32),
          pltpu.SemaphoreType.DMA,
      ],
  )
  def k(ids_hbm, table_hbm, out_hbm, idx_v, rows_v, sem):
    wid = lax.axis_index("s") * NC + lax.axis_index("c")
    w_base = wid * n_per_w

    def body(i, _):
      base = w_base + i * CHUNK
      pltpu.sync_copy(ids_hbm.at[pl.ds(base, CHUNK)], idx_v)
      pltpu.async_copy(table_hbm.at[idx_v], rows_v, sem).wait()
      pltpu.sync_copy(rows_v, out_hbm.at[pl.ds(base, CHUNK)])
      return ()

    lax.fori_loop(0, n_chunks, body, (), unroll=False)

  return k


def kernel(ids, emb_table):
  batch, hist = ids.shape
  ntok, d = emb_table.shape
  flat = ids.reshape(-1).astype(jnp.int32)
  out = _emb_kernel(flat.shape[0], d, 1024)(flat, emb_table)
  return out.reshape(batch, hist, d)


# speedup vs baseline: 1.1580x; 1.0131x over previous
import functools

import jax
import jax.numpy as jnp
from jax import lax
from jax.experimental import pallas as pl
from jax.experimental.pallas import tpu as pltpu
from jax.experimental.pallas import tpu_sc as plsc

NC = 2
NS = 16
NW = NC * NS


def _emb_kernel(B, D, CHUNK, NBUF):
  n_per_w = B // NW
  n_chunks = n_per_w // CHUNK
  mesh = plsc.VectorSubcoreMesh(
      core_axis_name="c", subcore_axis_name="s", num_cores=NC,
      num_subcores=NS)

  @functools.partial(
      pl.kernel,
      mesh=mesh,
      compiler_params=pltpu.CompilerParams(use_tc_tiling_on_sc=False),
      out_type=jax.ShapeDtypeStruct((B, D), jnp.float32),
      scratch_types=[
          pltpu.VMEM((NBUF, CHUNK), jnp.int32),
          pltpu.VMEM((NBUF, CHUNK, D), jnp.float32),
          [pltpu.SemaphoreType.DMA] * NBUF,
          [pltpu.SemaphoreType.DMA] * NBUF,
          [pltpu.SemaphoreType.DMA] * NBUF,
      ],
  )
  def k(ids_hbm, table_hbm, out_hbm, idx_v, rows_v, sem_i, sem_g, sem_o):
    wid = lax.axis_index("s") * NC + lax.axis_index("c")
    w_base = wid * n_per_w

    d_i, d_o = {}, {}
    for g in range(min(NBUF, n_chunks)):
      d_i[g] = pltpu.async_copy(
          ids_hbm.at[pl.ds(w_base + g * CHUNK, CHUNK)],
          idx_v.at[g % NBUF], sem_i[g % NBUF])
    for g in range(n_chunks):
      b = g % NBUF
      if g >= NBUF:
        d_o[g - NBUF].wait()
      d_i[g].wait()
      pltpu.async_copy(
          table_hbm.at[idx_v.at[b]], rows_v.at[b], sem_g[b]).wait()
      if g + NBUF < n_chunks:
        d_i[g + NBUF] = pltpu.async_copy(
            ids_hbm.at[pl.ds(w_base + (g + NBUF) * CHUNK, CHUNK)],
            idx_v.at[b], sem_i[b])
      d_o[g] = pltpu.async_copy(
          rows_v.at[b], out_hbm.at[pl.ds(w_base + g * CHUNK, CHUNK)],
          sem_o[b])
    for g in range(max(0, n_chunks - NBUF), n_chunks):
      d_o[g].wait()

  return k


def kernel(ids, emb_table):
  batch, hist = ids.shape
  ntok, d = emb_table.shape
  flat = ids.reshape(-1).astype(jnp.int32)
  out = _emb_kernel(flat.shape[0], d, 1600, 2)(flat, emb_table)
  return out.reshape(batch, hist, d)


# speedup vs baseline: 1.5683x; 1.3543x over previous
import functools

import jax
import jax.numpy as jnp
from jax import lax
from jax.experimental import pallas as pl
from jax.experimental.pallas import tpu as pltpu
from jax.experimental.pallas import tpu_sc as plsc

NC = 2
NS = 16
NW = NC * NS
LANES = 16


def _emb_kernel(HIST, BATCH, NTOK, D):
  BPW = BATCH // NW
  assert BATCH % NW == 0 and HIST % 2 == 0 and BPW % LANES == 0
  mesh = plsc.VectorSubcoreMesh(
      core_axis_name="c", subcore_axis_name="s", num_cores=NC,
      num_subcores=NS)

  @functools.partial(
      pl.kernel,
      mesh=mesh,
      compiler_params=pltpu.CompilerParams(
          use_tc_tiling_on_sc=False, needs_layout_passes=False),
      out_type=jax.ShapeDtypeStruct((HIST, D, BATCH), jnp.float32),
      scratch_types=[
          pltpu.VMEM((HIST, BPW), jnp.int32),
          pltpu.VMEM((BPW, D), jnp.float32),
          pltpu.VMEM((BPW, D), jnp.float32),
          pltpu.VMEM((D, BPW), jnp.float32),
          pltpu.VMEM((D, BPW), jnp.float32),
          pltpu.SemaphoreType.DMA,
          pltpu.SemaphoreType.DMA,
          pltpu.SemaphoreType.DMA,
          pltpu.SemaphoreType.DMA,
      ],
  )
  def k(ids_hbm, table_hbm, out_hbm, idx_all, rows0, rows1, trans0, trans1,
        sg0, sg1, ss0, ss1):
    wid = lax.axis_index("s") * NC + lax.axis_index("c")
    b0 = wid * BPW
    pltpu.sync_copy(ids_hbm.at[:, pl.ds(b0, BPW)], idx_all)
    iota = lax.iota(jnp.int32, LANES)

    def g_start(h, rows, sem):
      pltpu.async_copy(table_hbm.at[idx_all.at[h]], rows, sem)

    def g_wait(h, rows, sem):
      pltpu.make_async_copy(table_hbm.at[idx_all.at[h]], rows, sem).wait()

    def s_start(h, trans, sem):
      pltpu.async_copy(trans, out_hbm.at[h, :, pl.ds(b0, BPW)], sem)

    def s_wait(h, trans, sem):
      pltpu.make_async_copy(
          trans, out_hbm.at[h, :, pl.ds(b0, BPW)], sem).wait()

    def transpose(rows, trans):
      def tbody(d, carry):
        col = jnp.full((LANES,), 0, jnp.int32) + d
        for bb in range(BPW // LANES):
          v = plsc.load_gather(rows, [iota + (bb * LANES), col])
          trans[d, pl.ds(bb * LANES, LANES)] = v
        return carry
      lax.fori_loop(0, D, tbody, 0, unroll=False)

    P = HIST // 2
    g_start(0, rows0, sg0)
    g_wait(0, rows0, sg0)
    g_start(1, rows1, sg1)
    transpose(rows0, trans0)
    s_start(0, trans0, ss0)
    g_start(2, rows0, sg0)
    g_wait(1, rows1, sg1)
    transpose(rows1, trans1)
    s_start(1, trans1, ss1)

    def body(i, carry):
      h0 = 2 * i
      h1 = h0 + 1
      g_wait(h0, rows0, sg0)
      g_start(h1, rows1, sg1)
      s_wait(h0 - 2, trans0, ss0)
      transpose(rows0, trans0)
      s_start(h0, trans0, ss0)
      g_start(h0 + 2, rows0, sg0)
      g_wait(h1, rows1, sg1)
      s_wait(h1 - 2, trans1, ss1)
      transpose(rows1, trans1)
      s_start(h1, trans1, ss1)
      return carry

    lax.fori_loop(1, P - 1, body, 0, unroll=False)

    h0 = HIST - 2
    h1 = HIST - 1
    g_wait(h0, rows0, sg0)
    g_start(h1, rows1, sg1)
    s_wait(h0 - 2, trans0, ss0)
    transpose(rows0, trans0)
    s_start(h0, trans0, ss0)
    g_wait(h1, rows1, sg1)
    s_wait(h1 - 2, trans1, ss1)
    transpose(rows1, trans1)
    s_start(h1, trans1, ss1)
    s_wait(h0, trans0, ss0)
    s_wait(h1, trans1, ss1)

  return k


def kernel(ids, emb_table):
  batch, hist = ids.shape
  ntok, d = emb_table.shape
  ids_t = ids.astype(jnp.int32).T
  out_t = _emb_kernel(hist, batch, ntok, d)(ids_t, emb_table)
  return jnp.transpose(out_t, (2, 0, 1))


# speedup vs baseline: 2.6312x; 1.6778x over previous
import functools

import jax
import jax.numpy as jnp
from jax import lax
from jax.experimental import pallas as pl
from jax.experimental.pallas import tpu as pltpu
from jax.experimental.pallas import tpu_sc as plsc

NC = 2
NS = 16
NW = NC * NS
LANES = 16


def _emb_kernel(HIST, BATCH, NTOK, D):
  BPW = BATCH // NW
  assert BATCH % NW == 0 and HIST % 2 == 0 and BPW % LANES == 0
  mesh = plsc.VectorSubcoreMesh(
      core_axis_name="c", subcore_axis_name="s", num_cores=NC,
      num_subcores=NS)

  @functools.partial(
      pl.kernel,
      mesh=mesh,
      compiler_params=pltpu.CompilerParams(
          use_tc_tiling_on_sc=False, needs_layout_passes=False),
      out_type=jax.ShapeDtypeStruct((HIST, D, BATCH), jnp.float32),
      scratch_types=[
          pltpu.VMEM((HIST, BPW), jnp.int32),
          pltpu.VMEM((BPW, D), jnp.float32),
          pltpu.VMEM((BPW, D), jnp.float32),
          pltpu.VMEM((D, BPW + 1), jnp.float32),
          pltpu.VMEM((D, BPW + 1), jnp.float32),
          pltpu.SemaphoreType.DMA,
          pltpu.SemaphoreType.DMA,
          pltpu.SemaphoreType.DMA,
          pltpu.SemaphoreType.DMA,
      ],
  )
  def k(ids_hbm, table_hbm, out_hbm, idx_all, rows0, rows1, trans0, trans1,
        sg0, sg1, ss0, ss1):
    wid = lax.axis_index("s") * NC + lax.axis_index("c")
    b0 = wid * BPW
    pltpu.sync_copy(ids_hbm.at[:, pl.ds(b0, BPW)], idx_all)
    iota = lax.iota(jnp.int32, LANES)

    def g_start(h, rows, sem):
      pltpu.async_copy(table_hbm.at[idx_all.at[h]], rows, sem)

    def g_wait(h, rows, sem):
      pltpu.make_async_copy(table_hbm.at[idx_all.at[h]], rows, sem).wait()

    def s_start(h, trans, sem):
      pltpu.async_copy(
          trans.at[:, pl.ds(0, BPW)], out_hbm.at[h, :, pl.ds(b0, BPW)], sem)

    def s_wait(h, trans, sem):
      pltpu.make_async_copy(
          trans.at[:, pl.ds(0, BPW)], out_hbm.at[h, :, pl.ds(b0, BPW)],
          sem).wait()

    def transpose(rows, trans):
      def tbody(g, carry):
        base = jnp.full((LANES,), 0, jnp.int32) + g * LANES
        for j in range(LANES):
          b = g * LANES + j
          v1 = rows[b, pl.ds(0, LANES)]
          v2 = rows[b, pl.ds(LANES, LANES)]
          bvec = base + j
          plsc.store_scatter(trans, [iota, bvec], v1)
          plsc.store_scatter(trans, [iota + LANES, bvec], v2)
        return carry
      lax.fori_loop(0, BPW // LANES, tbody, 0, unroll=False)

    P = HIST // 2
    g_start(0, rows0, sg0)
    g_wait(0, rows0, sg0)
    g_start(1, rows1, sg1)
    transpose(rows0, trans0)
    s_start(0, trans0, ss0)
    g_start(2, rows0, sg0)
    g_wait(1, rows1, sg1)
    transpose(rows1, trans1)
    s_start(1, trans1, ss1)

    def body(i, carry):
      h0 = 2 * i
      h1 = h0 + 1
      g_wait(h0, rows0, sg0)
      g_start(h1, rows1, sg1)
      s_wait(h0 - 2, trans0, ss0)
      transpose(rows0, trans0)
      s_start(h0, trans0, ss0)
      g_start(h0 + 2, rows0, sg0)
      g_wait(h1, rows1, sg1)
      s_wait(h1 - 2, trans1, ss1)
      transpose(rows1, trans1)
      s_start(h1, trans1, ss1)
      return carry

    lax.fori_loop(1, P - 1, body, 0, unroll=False)

    h0 = HIST - 2
    h1 = HIST - 1
    g_wait(h0, rows0, sg0)
    g_start(h1, rows1, sg1)
    s_wait(h0 - 2, trans0, ss0)
    transpose(rows0, trans0)
    s_start(h0, trans0, ss0)
    g_wait(h1, rows1, sg1)
    s_wait(h1 - 2, trans1, ss1)
    transpose(rows1, trans1)
    s_start(h1, trans1, ss1)
    s_wait(h0, trans0, ss0)
    s_wait(h1, trans1, ss1)

  return k


def kernel(ids, emb_table):
  batch, hist = ids.shape
  ntok, d = emb_table.shape
  ids_t = ids.astype(jnp.int32).T
  out_t = _emb_kernel(hist, batch, ntok, d)(ids_t, emb_table)
  return jnp.transpose(out_t, (2, 0, 1))


# speedup vs baseline: 2.6741x; 1.0163x over previous
import functools

import jax
import jax.numpy as jnp
from jax import lax
from jax.experimental import pallas as pl
from jax.experimental.pallas import tpu as pltpu
from jax.experimental.pallas import tpu_sc as plsc

NC = 2
NS = 16
NW = NC * NS
LANES = 16


def _emb_kernel(HIST, BATCH, NTOK, D):
  BPW = BATCH // NW
  assert BATCH % NW == 0 and HIST % 2 == 0 and BPW % LANES == 0
  mesh = plsc.VectorSubcoreMesh(
      core_axis_name="c", subcore_axis_name="s", num_cores=NC,
      num_subcores=NS)

  @functools.partial(
      pl.kernel,
      mesh=mesh,
      compiler_params=pltpu.CompilerParams(
          use_tc_tiling_on_sc=False, needs_layout_passes=False),
      out_type=jax.ShapeDtypeStruct((HIST, D, BATCH), jnp.float32),
      scratch_types=[
          pltpu.VMEM((HIST, BPW), jnp.int32),
          pltpu.VMEM((BPW, D), jnp.float32),
          pltpu.VMEM((BPW, D), jnp.float32),
          pltpu.VMEM((D, BPW + 1), jnp.float32),
          pltpu.VMEM((D, BPW + 1), jnp.float32),
          pltpu.SemaphoreType.DMA,
          pltpu.SemaphoreType.DMA,
          pltpu.SemaphoreType.DMA,
          pltpu.SemaphoreType.DMA,
      ],
  )
  def k(ids_hbm, table_hbm, out_hbm, idx_all, rows0, rows1, trans0, trans1,
        sg0, sg1, ss0, ss1):
    wid = lax.axis_index("s") * NC + lax.axis_index("c")
    b0 = wid * BPW
    pltpu.sync_copy(ids_hbm.at[:, pl.ds(b0, BPW)], idx_all)
    iota = lax.iota(jnp.int32, LANES)

    def scale_body(h, carry):
      for j in range(BPW // LANES):
        idx_all[h, pl.ds(j * LANES, LANES)] = lax.shift_left(
            idx_all[h, pl.ds(j * LANES, LANES)], 2)
      return carry
    lax.fori_loop(0, HIST, scale_body, 0, unroll=False)

    def g_start(h, rows, sem):
      pltpu.async_copy(table_hbm.at[idx_all.at[h]], rows, sem)

    def g_wait(h, rows, sem):
      pltpu.make_async_copy(table_hbm.at[idx_all.at[h]], rows, sem).wait()

    def s_start(h, trans, sem):
      pltpu.async_copy(
          trans.at[:, pl.ds(0, BPW)], out_hbm.at[h, :, pl.ds(b0, BPW)], sem)

    def s_wait(h, trans, sem):
      pltpu.make_async_copy(
          trans.at[:, pl.ds(0, BPW)], out_hbm.at[h, :, pl.ds(b0, BPW)],
          sem).wait()

    def transpose(rows, trans):
      def tbody(g, carry):
        base = jnp.full((LANES,), 0, jnp.int32) + g * LANES
        for j in range(LANES):
          b = g * LANES + j
          v1 = rows[b, pl.ds(0, LANES)]
          v2 = rows[b, pl.ds(LANES, LANES)]
          bvec = base + j
          plsc.store_scatter(trans, [iota, bvec], v1)
          plsc.store_scatter(trans, [iota + LANES, bvec], v2)
        return carry
      lax.fori_loop(0, BPW // LANES, tbody, 0, unroll=False)

    P = HIST // 2
    g_start(0, rows0, sg0)
    g_wait(0, rows0, sg0)
    g_start(1, rows1, sg1)
    transpose(rows0, trans0)
    s_start(0, trans0, ss0)
    g_start(2, rows0, sg0)
    g_wait(1, rows1, sg1)
    transpose(rows1, trans1)
    s_start(1, trans1, ss1)

    def body(i, carry):
      h0 = 2 * i
      h1 = h0 + 1
      g_wait(h0, rows0, sg0)
      g_start(h1, rows1, sg1)
      s_wait(h0 - 2, trans0, ss0)
      transpose(rows0, trans0)
      s_start(h0, trans0, ss0)
      g_start(h0 + 2, rows0, sg0)
      g_wait(h1, rows1, sg1)
      s_wait(h1 - 2, trans1, ss1)
      transpose(rows1, trans1)
      s_start(h1, trans1, ss1)
      return carry

    lax.fori_loop(1, P - 1, body, 0, unroll=False)

    h0 = HIST - 2
    h1 = HIST - 1
    g_wait(h0, rows0, sg0)
    g_start(h1, rows1, sg1)
    s_wait(h0 - 2, trans0, ss0)
    transpose(rows0, trans0)
    s_start(h0, trans0, ss0)
    g_wait(h1, rows1, sg1)
    s_wait(h1 - 2, trans1, ss1)
    transpose(rows1, trans1)
    s_start(h1, trans1, ss1)
    s_wait(h0, trans0, ss0)
    s_wait(h1, trans1, ss1)

  return k


def kernel(ids, emb_table):
  batch, hist = ids.shape
  ntok, d = emb_table.shape
  ids_t = ids.astype(jnp.int32).T
  tpad = jnp.pad(emb_table, ((0, 0), (0, 128 - d)))
  tview = tpad.reshape(ntok * (128 // d), d)
  out_t = _emb_kernel(hist, batch, ntok, d)(ids_t, tview)
  return jnp.transpose(out_t, (2, 0, 1))


# speedup vs baseline: 3.0738x; 1.1495x over previous
import functools

import jax
import jax.numpy as jnp
from jax import lax
from jax.experimental import pallas as pl
from jax.experimental.pallas import tpu as pltpu
from jax.experimental.pallas import tpu_sc as plsc

NC = 2
NS = 16
NW = NC * NS
LANES = 16


def _emb_kernel(HIST, BATCH, NTOK, D):
  BPW = BATCH // NW
  assert BATCH % NW == 0 and HIST % 2 == 0 and BPW % LANES == 0
  mesh = plsc.VectorSubcoreMesh(
      core_axis_name="c", subcore_axis_name="s", num_cores=NC,
      num_subcores=NS)

  @functools.partial(
      pl.kernel,
      mesh=mesh,
      compiler_params=pltpu.CompilerParams(
          use_tc_tiling_on_sc=False, needs_layout_passes=False),
      out_type=jax.ShapeDtypeStruct(
          (HIST, D // 8, BATCH // 128, 8, 128), jnp.float32),
      scratch_types=[
          pltpu.VMEM((HIST, BPW), jnp.int32),
          pltpu.VMEM((BPW, D), jnp.float32),
          pltpu.VMEM((BPW, D), jnp.float32),
          pltpu.VMEM((D, BPW + 1), jnp.float32),
          pltpu.VMEM((D, BPW + 1), jnp.float32),
          pltpu.SemaphoreType.DMA,
          pltpu.SemaphoreType.DMA,
          pltpu.SemaphoreType.DMA,
          pltpu.SemaphoreType.DMA,
      ],
  )
  def k(ids_hbm, table_hbm, out_hbm, idx_all, rows0, rows1, trans0, trans1,
        sg0, sg1, ss0, ss1):
    wid = lax.axis_index("s") * NC + lax.axis_index("c")
    b0 = wid * BPW
    pltpu.sync_copy(ids_hbm.at[:, pl.ds(b0, BPW)], idx_all)
    iota = lax.iota(jnp.int32, LANES)

    def scale_body(h, carry):
      for j in range(BPW // LANES):
        idx_all[h, pl.ds(j * LANES, LANES)] = lax.shift_left(
            idx_all[h, pl.ds(j * LANES, LANES)], 2)
      return carry
    lax.fori_loop(0, HIST, scale_body, 0, unroll=False)

    def g_start(h, rows, sem):
      pltpu.async_copy(table_hbm.at[idx_all.at[h]], rows, sem)

    def g_wait(h, rows, sem):
      pltpu.make_async_copy(table_hbm.at[idx_all.at[h]], rows, sem).wait()

    tc0 = wid * (BPW // 128)

    def s_start(h, trans, sem):
      for tr in range(D // 8):
        for tc in range(BPW // 128):
          pltpu.async_copy(
              trans.at[pl.ds(tr * 8, 8), pl.ds(tc * 128, 128)],
              out_hbm.at[h, tr, tc0 + tc], sem)

    def s_wait(h, trans, sem):
      for tr in range(D // 8):
        for tc in range(BPW // 128):
          pltpu.make_async_copy(
              trans.at[pl.ds(tr * 8, 8), pl.ds(tc * 128, 128)],
              out_hbm.at[h, tr, tc0 + tc], sem).wait()

    def transpose(rows, trans):
      def tbody(g, carry):
        base = jnp.full((LANES,), 0, jnp.int32) + g * LANES
        for j in range(LANES):
          b = g * LANES + j
          v1 = rows[b, pl.ds(0, LANES)]
          v2 = rows[b, pl.ds(LANES, LANES)]
          bvec = base + j
          plsc.store_scatter(trans, [iota, bvec], v1)
          plsc.store_scatter(trans, [iota + LANES, bvec], v2)
        return carry
      lax.fori_loop(0, BPW // LANES, tbody, 0, unroll=False)

    P = HIST // 2
    g_start(0, rows0, sg0)
    g_wait(0, rows0, sg0)
    g_start(1, rows1, sg1)
    transpose(rows0, trans0)
    s_start(0, trans0, ss0)
    g_start(2, rows0, sg0)
    g_wait(1, rows1, sg1)
    transpose(rows1, trans1)
    s_start(1, trans1, ss1)

    def body(i, carry):
      h0 = 2 * i
      h1 = h0 + 1
      g_wait(h0, rows0, sg0)
      g_start(h1, rows1, sg1)
      s_wait(h0 - 2, trans0, ss0)
      transpose(rows0, trans0)
      s_start(h0, trans0, ss0)
      g_start(h0 + 2, rows0, sg0)
      g_wait(h1, rows1, sg1)
      s_wait(h1 - 2, trans1, ss1)
      transpose(rows1, trans1)
      s_start(h1, trans1, ss1)
      return carry

    lax.fori_loop(1, P - 1, body, 0, unroll=False)

    h0 = HIST - 2
    h1 = HIST - 1
    g_wait(h0, rows0, sg0)
    g_start(h1, rows1, sg1)
    s_wait(h0 - 2, trans0, ss0)
    transpose(rows0, trans0)
    s_start(h0, trans0, ss0)
    g_wait(h1, rows1, sg1)
    s_wait(h1 - 2, trans1, ss1)
    transpose(rows1, trans1)
    s_start(h1, trans1, ss1)
    s_wait(h0, trans0, ss0)
    s_wait(h1, trans1, ss1)

  return k


def kernel(ids, emb_table):
  batch, hist = ids.shape
  ntok, d = emb_table.shape
  ids_t = ids.astype(jnp.int32).T
  tpad = jnp.pad(emb_table, ((0, 0), (0, 128 - d)))
  tview = tpad.reshape(ntok * (128 // d), d)
  out5 = _emb_kernel(hist, batch, ntok, d)(ids_t, tview)
  return jnp.transpose(out5, (2, 4, 0, 1, 3)).reshape(batch, hist, d)
